# baseline (speedup 1.0000x reference)
import functools

import jax
import jax.numpy as jnp
from jax import lax
from jax.experimental import pallas as pl
from jax.experimental.pallas import tpu as pltpu
from jax.experimental.pallas import tpu_sc as plsc

_NUM_CORES = 2
_NUM_SUBCORES = 16
_NUM_WORKERS = _NUM_CORES * _NUM_SUBCORES
_GROUP = 16


@functools.lru_cache(maxsize=None)
def _make_kernel(V, D, B):
    b_per_w = B // _NUM_WORKERS
    n_groups = b_per_w // _GROUP
    mesh = plsc.VectorSubcoreMesh(
        core_axis_name="c",
        subcore_axis_name="s",
        num_cores=_NUM_CORES,
        num_subcores=_NUM_SUBCORES,
    )

    @functools.partial(
        pl.kernel,
        mesh=mesh,
        out_type=jax.ShapeDtypeStruct((B, D), jnp.float32),
        scratch_types=[
            pltpu.VMEM((b_per_w,), jnp.int32),
            pltpu.VMEM((b_per_w, D), jnp.float32),
            pltpu.SemaphoreType.DMA,
        ],
    )
    def gather_kernel(idx_hbm, table_hbm, out_hbm, idx_v, rows_v, sem):
        wid = lax.axis_index("s") * _NUM_CORES + lax.axis_index("c")
        base = wid * b_per_w
        pltpu.sync_copy(idx_hbm.at[pl.ds(base, b_per_w)], idx_v)

        def step(g, _):
            iv = idx_v[pl.ds(g * _GROUP, _GROUP)]
            copies = []
            for u in range(_GROUP):
                copies.append(
                    pltpu.async_copy(
                        table_hbm.at[iv[u]],
                        rows_v.at[g * _GROUP + u],
                        sem,
                    )
                )
            for c in copies:
                c.wait()
            return 0

        lax.fori_loop(0, n_groups, step, 0)
        pltpu.sync_copy(rows_v, out_hbm.at[pl.ds(base, b_per_w)])

    return gather_kernel


def kernel(provider_ids, table):
    (B,) = provider_ids.shape
    V, D = table.shape
    idx = provider_ids.astype(jnp.int32)
    return _make_kernel(V, D, B)(idx, table)
